# baseline (device time: 24935 ns/iter reference)
import jax
import jax.numpy as jnp
from jax import lax
from jax.experimental import pallas as pl
from jax.experimental.pallas import tpu as pltpu

N_DEV = 8
B = 2
S_PER = 128
S_GLOB = N_DEV * S_PER
HQ = 4
DH = 64
G = HQ * B
S_NEAR = 3 * S_PER
LOCAL_WINDOW = 128
GLOBAL_PREFIX = 32
MIDS = (2, 3, 4, 5, 6)


def kernel(x, Wq, K_ext, V_ext, Wo):
    d_model = x.shape[-1]

    def body(x_ref, wq_ref, k_ref, v_ref, wo_ref, out_ref,
             nb_ref, q32_ref, k0_ref, part_ref, parts0_ref,
             num32_ref, den32_ref,
             nbr_r_send, nbr_r_recv, nbr_l_send, nbr_l_recv,
             q32_send, q32_recv, k0_send, k0_recv,
             part_send, part_recv):
        my = lax.axis_index("i")
        left = lax.rem(my - 1 + N_DEV, N_DEV)
        right = lax.rem(my + 1, N_DEV)
        is_zero = my == 0
        is_mid = (my >= 2) & (my <= 6)

        barrier_sem = pltpu.get_barrier_semaphore()
        for nbr in (left, right):
            pl.semaphore_signal(
                barrier_sem, inc=1,
                device_id=(nbr,), device_id_type=pl.DeviceIdType.MESH,
            )

        @pl.when(is_zero)
        def _():
            for d in MIDS:
                pl.semaphore_signal(
                    barrier_sem, inc=1,
                    device_id=(d,), device_id_type=pl.DeviceIdType.MESH,
                )

        @pl.when(is_mid)
        def _():
            pl.semaphore_signal(
                barrier_sem, inc=1,
                device_id=(0,), device_id_type=pl.DeviceIdType.MESH,
            )

        pl.semaphore_wait(barrier_sem, 2)

        @pl.when(is_zero)
        def _():
            pl.semaphore_wait(barrier_sem, len(MIDS))

        @pl.when(is_mid)
        def _():
            pl.semaphore_wait(barrier_sem, 1)

        for j in range(HQ):
            nb_ref[pl.ds(j * B, B), pl.ds(S_PER, S_PER), :] = (
                k_ref[:, :, j, :].astype(jnp.bfloat16))
            nb_ref[pl.ds(G + j * B, B), pl.ds(S_PER, S_PER), :] = (
                v_ref[:, :, j, :].astype(jnp.bfloat16))

        @pl.when(jnp.logical_not(is_mid))
        def _():
            k0_ref[:] = jnp.zeros((2 * G, GLOBAL_PREFIX, DH), jnp.bfloat16)

        send_r = pltpu.make_async_remote_copy(
            src_ref=nb_ref.at[:, pl.ds(S_PER, S_PER), :],
            dst_ref=nb_ref.at[:, pl.ds(0, S_PER), :],
            send_sem=nbr_r_send.at[0], recv_sem=nbr_r_recv.at[0],
            device_id=(right,), device_id_type=pl.DeviceIdType.MESH,
        )
        send_l = pltpu.make_async_remote_copy(
            src_ref=nb_ref.at[:, pl.ds(S_PER, S_PER), :],
            dst_ref=nb_ref.at[:, pl.ds(2 * S_PER, S_PER), :],
            send_sem=nbr_l_send.at[0], recv_sem=nbr_l_recv.at[0],
            device_id=(left,), device_id_type=pl.DeviceIdType.MESH,
        )
        send_r.start()
        send_l.start()

        xb = x_ref[:].astype(jnp.bfloat16)
        wqb = wq_ref[:].astype(jnp.bfloat16)

        q32_rdmas = []
        k0_rdmas = []
        for i, d in enumerate(MIDS):
            q32_rdmas.append(pltpu.make_async_remote_copy(
                src_ref=q32_ref, dst_ref=q32_ref,
                send_sem=q32_send.at[i], recv_sem=q32_recv.at[0],
                device_id=(d,), device_id_type=pl.DeviceIdType.MESH,
            ))
            k0_rdmas.append(pltpu.make_async_remote_copy(
                src_ref=nb_ref.at[:, pl.ds(S_PER, GLOBAL_PREFIX), :],
                dst_ref=k0_ref,
                send_sem=k0_send.at[i], recv_sem=k0_recv.at[0],
                device_id=(d,), device_id_type=pl.DeviceIdType.MESH,
            ))

        @pl.when(is_zero)
        def _():
            q32 = lax.dot_general(
                x_ref[:, :GLOBAL_PREFIX, :].astype(jnp.bfloat16), wqb,
                (((2,), (0,)), ((), ())),
                preferred_element_type=jnp.float32,
            )
            q32_ref[:] = jnp.concatenate(
                [q32[:, :, h * DH:(h + 1) * DH] for h in range(HQ)], axis=0
            ).astype(jnp.bfloat16)
            for r in q32_rdmas:
                r.start()
            for r in k0_rdmas:
                r.start()

        q = lax.dot_general(
            xb, wqb, (((2,), (0,)), ((), ())),
            preferred_element_type=jnp.float32,
        )
        qall = jnp.concatenate(
            [q[:, :, h * DH:(h + 1) * DH] for h in range(HQ)], axis=0
        ).astype(jnp.bfloat16)

        part_rdma = pltpu.make_async_remote_copy(
            src_ref=part_ref, dst_ref=parts0_ref.at[my - 2],
            send_sem=part_send.at[0], recv_sem=part_recv.at[my - 2],
            device_id=(0,), device_id_type=pl.DeviceIdType.MESH,
        )

        @pl.when(is_mid)
        def _():
            q32_rdmas[0].wait_recv()
            kb_own = nb_ref[0:G, pl.ds(S_PER, S_PER), :]
            vb_own = nb_ref[G:2 * G, pl.ds(S_PER, S_PER), :]
            s32 = lax.dot_general(
                q32_ref[:], kb_own, (((2,), (2,)), ((0,), (0,))),
                preferred_element_type=jnp.float32,
            ) * 0.125
            e32 = jnp.exp(s32)
            p_num = lax.dot_general(
                e32.astype(jnp.bfloat16), vb_own,
                (((2,), (1,)), ((0,), (0,))),
                preferred_element_type=jnp.float32,
            )
            p_den = jnp.sum(e32, axis=-1, keepdims=True)
            part_ref[0] = p_num.astype(jnp.bfloat16)
            part_ref[1] = jnp.broadcast_to(
                p_den, (G, GLOBAL_PREFIX, DH)).astype(jnp.bfloat16)
            part_rdma.start()

        send_r.wait_recv()
        send_l.wait_recv()

        qi_glob = (lax.broadcasted_iota(jnp.int32, (S_PER, S_NEAR), 0)
                   + my * S_PER)
        fake_ki = (lax.broadcasted_iota(jnp.int32, (S_PER, S_NEAR), 1)
                   + (my - 1) * S_PER)
        real_ki = lax.rem(fake_ki + S_GLOB, S_GLOB)
        mask = ((jnp.abs(qi_glob - real_ki) <= LOCAL_WINDOW)
                | (real_ki < GLOBAL_PREFIX) | (qi_glob < GLOBAL_PREFIX))
        maskf = mask.astype(jnp.float32)[None, :, :]

        kb = nb_ref[0:G]
        vb = nb_ref[G:2 * G]
        s = lax.dot_general(
            qall, kb, (((2,), (2,)), ((0,), (0,))),
            preferred_element_type=jnp.float32,
        ) * 0.125
        e = jnp.exp(s) * maskf
        num = lax.dot_general(
            e.astype(jnp.bfloat16), vb, (((2,), (1,)), ((0,), (0,))),
            preferred_element_type=jnp.float32,
        )
        den = jnp.sum(e, axis=-1, keepdims=True)

        @pl.when(is_mid)
        def _():
            k0_rdmas[0].wait_recv()

        sb = lax.dot_general(
            qall, k0_ref[0:G], (((2,), (2,)), ((0,), (0,))),
            preferred_element_type=jnp.float32,
        ) * 0.125
        eb = jnp.where(is_mid, jnp.exp(sb), 0.0)
        num = num + lax.dot_general(
            eb.astype(jnp.bfloat16), k0_ref[G:2 * G],
            (((2,), (1,)), ((0,), (0,))),
            preferred_element_type=jnp.float32,
        )
        den = den + jnp.sum(eb, axis=-1, keepdims=True)

        @pl.when(is_zero)
        def _():
            for i in range(len(MIDS)):
                part_recv_desc = pltpu.make_async_remote_copy(
                    src_ref=part_ref, dst_ref=parts0_ref.at[i],
                    send_sem=part_send.at[0], recv_sem=part_recv.at[i],
                    device_id=(0,), device_id_type=pl.DeviceIdType.MESH,
                )
                part_recv_desc.wait_recv()
            n32 = parts0_ref[0, 0].astype(jnp.float32)
            d32 = parts0_ref[0, 1, :, :, 0:1].astype(jnp.float32)
            for i in range(1, len(MIDS)):
                n32 = n32 + parts0_ref[i, 0].astype(jnp.float32)
                d32 = d32 + parts0_ref[i, 1, :, :, 0:1].astype(jnp.float32)
            num32_ref[:] = n32
            den32_ref[:] = d32

        @pl.when(jnp.logical_not(is_zero))
        def _():
            num32_ref[:] = jnp.zeros((G, GLOBAL_PREFIX, DH), jnp.float32)
            den32_ref[:] = jnp.zeros((G, GLOBAL_PREFIX, 1), jnp.float32)

        n = jnp.concatenate(
            [num[:, :GLOBAL_PREFIX, :] + num32_ref[:],
             num[:, GLOBAL_PREFIX:, :]], axis=1)
        d = jnp.concatenate(
            [den[:, :GLOBAL_PREFIX, :] + den32_ref[:],
             den[:, GLOBAL_PREFIX:, :]], axis=1)

        ctx = (n / d).astype(jnp.bfloat16)
        acc = jnp.zeros((B, S_PER, d_model), dtype=jnp.float32)
        for h in range(HQ):
            woh = wo_ref[h * DH:(h + 1) * DH, :].astype(jnp.bfloat16)
            acc = acc + lax.dot_general(
                ctx[h * B:(h + 1) * B], woh, (((2,), (0,)), ((), ())),
                preferred_element_type=jnp.float32,
            )
        out_ref[:] = acc

        send_r.wait_send()
        send_l.wait_send()

        @pl.when(is_zero)
        def _():
            for r in q32_rdmas:
                r.wait_send()
            for r in k0_rdmas:
                r.wait_send()

        @pl.when(is_mid)
        def _():
            part_rdma.wait_send()

    out_shape = jax.ShapeDtypeStruct((B, S_PER, d_model), jnp.float32)
    return pl.pallas_call(
        body,
        out_shape=out_shape,
        in_specs=[pl.BlockSpec(memory_space=pltpu.VMEM)] * 5,
        out_specs=pl.BlockSpec(memory_space=pltpu.VMEM),
        scratch_shapes=[
            pltpu.VMEM((2 * G, S_NEAR, DH), jnp.bfloat16),
            pltpu.VMEM((G, GLOBAL_PREFIX, DH), jnp.bfloat16),
            pltpu.VMEM((2 * G, GLOBAL_PREFIX, DH), jnp.bfloat16),
            pltpu.VMEM((2, G, GLOBAL_PREFIX, DH), jnp.bfloat16),
            pltpu.VMEM((len(MIDS), 2, G, GLOBAL_PREFIX, DH),
                       jnp.bfloat16),
            pltpu.VMEM((G, GLOBAL_PREFIX, DH), jnp.float32),
            pltpu.VMEM((G, GLOBAL_PREFIX, 1), jnp.float32),
            pltpu.SemaphoreType.DMA((1,)),
            pltpu.SemaphoreType.DMA((1,)),
            pltpu.SemaphoreType.DMA((1,)),
            pltpu.SemaphoreType.DMA((1,)),
            pltpu.SemaphoreType.DMA((len(MIDS),)),
            pltpu.SemaphoreType.DMA((1,)),
            pltpu.SemaphoreType.DMA((len(MIDS),)),
            pltpu.SemaphoreType.DMA((1,)),
            pltpu.SemaphoreType.DMA((1,)),
            pltpu.SemaphoreType.DMA((len(MIDS),)),
        ],
        compiler_params=pltpu.CompilerParams(collective_id=0),
    )(x, Wq, K_ext, V_ext, Wo)


# device time: 22825 ns/iter; 1.0924x vs baseline; 1.0924x over previous
import jax
import jax.numpy as jnp
from jax import lax
from jax.experimental import pallas as pl
from jax.experimental.pallas import tpu as pltpu

N_DEV = 8
B = 2
S_PER = 128
S_GLOB = N_DEV * S_PER
HQ = 4
DH = 64
G = HQ * B
S_NEAR = 3 * S_PER
LOCAL_WINDOW = 128
GLOBAL_PREFIX = 32
MIDS = (2, 3, 4, 5, 6)


def kernel(x, Wq, K_ext, V_ext, Wo):
    d_model = x.shape[-1]

    def body(x_ref, wq_ref, k_ref, v_ref, wo_ref, out_ref,
             nb_ref, q32_ref, k0_ref, part_ref, parts0_ref,
             num32_ref, den32_ref,
             nbr_r_send, nbr_r_recv, nbr_l_send, nbr_l_recv,
             q32_send, q32_recv, k0_send, k0_recv,
             part_send, part_recv):
        my = lax.axis_index("i")
        left = lax.rem(my - 1 + N_DEV, N_DEV)
        right = lax.rem(my + 1, N_DEV)
        is_zero = my == 0
        is_mid = (my >= 2) & (my <= 6)

        barrier_sem = pltpu.get_barrier_semaphore()
        for nbr in (left, right):
            pl.semaphore_signal(
                barrier_sem, inc=1,
                device_id=(nbr,), device_id_type=pl.DeviceIdType.MESH,
            )

        @pl.when(is_zero)
        def _():
            for d in MIDS:
                pl.semaphore_signal(
                    barrier_sem, inc=1,
                    device_id=(d,), device_id_type=pl.DeviceIdType.MESH,
                )

        @pl.when(is_mid)
        def _():
            pl.semaphore_signal(
                barrier_sem, inc=1,
                device_id=(0,), device_id_type=pl.DeviceIdType.MESH,
            )

        pl.semaphore_wait(barrier_sem, 2)

        @pl.when(is_zero)
        def _():
            pl.semaphore_wait(barrier_sem, len(MIDS))

        @pl.when(is_mid)
        def _():
            pl.semaphore_wait(barrier_sem, 1)

        for j in range(HQ):
            nb_ref[pl.ds(j * B, B), pl.ds(S_PER, S_PER), :] = (
                k_ref[:, :, j, :].astype(jnp.bfloat16))
            nb_ref[pl.ds(G + j * B, B), pl.ds(S_PER, S_PER), :] = (
                v_ref[:, :, j, :].astype(jnp.bfloat16))

        @pl.when(jnp.logical_not(is_mid))
        def _():
            k0_ref[:] = jnp.zeros((2 * G, GLOBAL_PREFIX, DH), jnp.bfloat16)

        send_r = pltpu.make_async_remote_copy(
            src_ref=nb_ref.at[:, pl.ds(S_PER, S_PER), :],
            dst_ref=nb_ref.at[:, pl.ds(0, S_PER), :],
            send_sem=nbr_r_send.at[0], recv_sem=nbr_r_recv.at[0],
            device_id=(right,), device_id_type=pl.DeviceIdType.MESH,
        )
        send_l = pltpu.make_async_remote_copy(
            src_ref=nb_ref.at[:, pl.ds(S_PER, S_PER), :],
            dst_ref=nb_ref.at[:, pl.ds(2 * S_PER, S_PER), :],
            send_sem=nbr_l_send.at[0], recv_sem=nbr_l_recv.at[0],
            device_id=(left,), device_id_type=pl.DeviceIdType.MESH,
        )
        xb = x_ref[:].astype(jnp.bfloat16)
        wqb = wq_ref[:].astype(jnp.bfloat16)

        q32_rdmas = []
        k0_rdmas = []
        for i, d in enumerate(MIDS):
            q32_rdmas.append(pltpu.make_async_remote_copy(
                src_ref=q32_ref, dst_ref=q32_ref,
                send_sem=q32_send.at[i], recv_sem=q32_recv.at[0],
                device_id=(d,), device_id_type=pl.DeviceIdType.MESH,
            ))
            k0_rdmas.append(pltpu.make_async_remote_copy(
                src_ref=nb_ref.at[:, pl.ds(S_PER, GLOBAL_PREFIX), :],
                dst_ref=k0_ref,
                send_sem=k0_send.at[i], recv_sem=k0_recv.at[0],
                device_id=(d,), device_id_type=pl.DeviceIdType.MESH,
            ))

        @pl.when(is_zero)
        def _():
            q32 = lax.dot_general(
                x_ref[:, :GLOBAL_PREFIX, :].astype(jnp.bfloat16), wqb,
                (((2,), (0,)), ((), ())),
                preferred_element_type=jnp.float32,
            )
            q32_ref[:] = jnp.concatenate(
                [q32[:, :, h * DH:(h + 1) * DH] for h in range(HQ)], axis=0
            ).astype(jnp.bfloat16)
            for r in q32_rdmas:
                r.start()
            for r in k0_rdmas:
                r.start()

        send_r.start()
        send_l.start()

        q = lax.dot_general(
            xb, wqb, (((2,), (0,)), ((), ())),
            preferred_element_type=jnp.float32,
        )
        qall = jnp.concatenate(
            [q[:, :, h * DH:(h + 1) * DH] for h in range(HQ)], axis=0
        ).astype(jnp.bfloat16)

        part_rdma = pltpu.make_async_remote_copy(
            src_ref=part_ref, dst_ref=parts0_ref.at[my - 2],
            send_sem=part_send.at[0], recv_sem=part_recv.at[my - 2],
            device_id=(0,), device_id_type=pl.DeviceIdType.MESH,
        )

        @pl.when(is_mid)
        def _():
            q32_rdmas[0].wait_recv()
            kb_own = nb_ref[0:G, pl.ds(S_PER, S_PER), :]
            vb_own = nb_ref[G:2 * G, pl.ds(S_PER, S_PER), :]
            s32 = lax.dot_general(
                q32_ref[:], kb_own, (((2,), (2,)), ((0,), (0,))),
                preferred_element_type=jnp.float32,
            ) * 0.125
            e32 = jnp.exp(s32)
            p_num = lax.dot_general(
                e32.astype(jnp.bfloat16), vb_own,
                (((2,), (1,)), ((0,), (0,))),
                preferred_element_type=jnp.float32,
            )
            p_den = jnp.sum(e32, axis=-1, keepdims=True)
            part_ref[0] = p_num.astype(jnp.bfloat16)
            part_ref[1] = jnp.broadcast_to(
                p_den, (G, GLOBAL_PREFIX, DH)).astype(jnp.bfloat16)
            part_rdma.start()

        @pl.when(is_mid)
        def _():
            k0_rdmas[0].wait_recv()

        sb = lax.dot_general(
            qall, k0_ref[0:G], (((2,), (2,)), ((0,), (0,))),
            preferred_element_type=jnp.float32,
        ) * 0.125
        eb = jnp.where(is_mid, jnp.exp(sb), 0.0)
        num = lax.dot_general(
            eb.astype(jnp.bfloat16), k0_ref[G:2 * G],
            (((2,), (1,)), ((0,), (0,))),
            preferred_element_type=jnp.float32,
        )
        den = jnp.sum(eb, axis=-1, keepdims=True)

        send_r.wait_recv()
        send_l.wait_recv()

        qi_glob = (lax.broadcasted_iota(jnp.int32, (S_PER, S_NEAR), 0)
                   + my * S_PER)
        fake_ki = (lax.broadcasted_iota(jnp.int32, (S_PER, S_NEAR), 1)
                   + (my - 1) * S_PER)
        real_ki = lax.rem(fake_ki + S_GLOB, S_GLOB)
        mask = ((jnp.abs(qi_glob - real_ki) <= LOCAL_WINDOW)
                | (real_ki < GLOBAL_PREFIX) | (qi_glob < GLOBAL_PREFIX))
        maskf = mask.astype(jnp.float32)[None, :, :]

        kb = nb_ref[0:G]
        vb = nb_ref[G:2 * G]
        s = lax.dot_general(
            qall, kb, (((2,), (2,)), ((0,), (0,))),
            preferred_element_type=jnp.float32,
        ) * 0.125
        e = jnp.exp(s) * maskf
        num = num + lax.dot_general(
            e.astype(jnp.bfloat16), vb, (((2,), (1,)), ((0,), (0,))),
            preferred_element_type=jnp.float32,
        )
        den = den + jnp.sum(e, axis=-1, keepdims=True)

        @pl.when(is_zero)
        def _():
            for i in range(len(MIDS)):
                part_recv_desc = pltpu.make_async_remote_copy(
                    src_ref=part_ref, dst_ref=parts0_ref.at[i],
                    send_sem=part_send.at[0], recv_sem=part_recv.at[i],
                    device_id=(0,), device_id_type=pl.DeviceIdType.MESH,
                )
                part_recv_desc.wait_recv()
            n32 = parts0_ref[0, 0].astype(jnp.float32)
            d32 = parts0_ref[0, 1, :, :, 0:1].astype(jnp.float32)
            for i in range(1, len(MIDS)):
                n32 = n32 + parts0_ref[i, 0].astype(jnp.float32)
                d32 = d32 + parts0_ref[i, 1, :, :, 0:1].astype(jnp.float32)
            num32_ref[:] = n32
            den32_ref[:] = d32

        @pl.when(jnp.logical_not(is_zero))
        def _():
            num32_ref[:] = jnp.zeros((G, GLOBAL_PREFIX, DH), jnp.float32)
            den32_ref[:] = jnp.zeros((G, GLOBAL_PREFIX, 1), jnp.float32)

        n = jnp.concatenate(
            [num[:, :GLOBAL_PREFIX, :] + num32_ref[:],
             num[:, GLOBAL_PREFIX:, :]], axis=1)
        d = jnp.concatenate(
            [den[:, :GLOBAL_PREFIX, :] + den32_ref[:],
             den[:, GLOBAL_PREFIX:, :]], axis=1)

        ctx = (n / d).astype(jnp.bfloat16)
        acc = jnp.zeros((B, S_PER, d_model), dtype=jnp.float32)
        for h in range(HQ):
            woh = wo_ref[h * DH:(h + 1) * DH, :].astype(jnp.bfloat16)
            acc = acc + lax.dot_general(
                ctx[h * B:(h + 1) * B], woh, (((2,), (0,)), ((), ())),
                preferred_element_type=jnp.float32,
            )
        out_ref[:] = acc

        send_r.wait_send()
        send_l.wait_send()

        @pl.when(is_zero)
        def _():
            for r in q32_rdmas:
                r.wait_send()
            for r in k0_rdmas:
                r.wait_send()

        @pl.when(is_mid)
        def _():
            part_rdma.wait_send()

    out_shape = jax.ShapeDtypeStruct((B, S_PER, d_model), jnp.float32)
    return pl.pallas_call(
        body,
        out_shape=out_shape,
        in_specs=[pl.BlockSpec(memory_space=pltpu.VMEM)] * 5,
        out_specs=pl.BlockSpec(memory_space=pltpu.VMEM),
        scratch_shapes=[
            pltpu.VMEM((2 * G, S_NEAR, DH), jnp.bfloat16),
            pltpu.VMEM((G, GLOBAL_PREFIX, DH), jnp.bfloat16),
            pltpu.VMEM((2 * G, GLOBAL_PREFIX, DH), jnp.bfloat16),
            pltpu.VMEM((2, G, GLOBAL_PREFIX, DH), jnp.bfloat16),
            pltpu.VMEM((len(MIDS), 2, G, GLOBAL_PREFIX, DH),
                       jnp.bfloat16),
            pltpu.VMEM((G, GLOBAL_PREFIX, DH), jnp.float32),
            pltpu.VMEM((G, GLOBAL_PREFIX, 1), jnp.float32),
            pltpu.SemaphoreType.DMA((1,)),
            pltpu.SemaphoreType.DMA((1,)),
            pltpu.SemaphoreType.DMA((1,)),
            pltpu.SemaphoreType.DMA((1,)),
            pltpu.SemaphoreType.DMA((len(MIDS),)),
            pltpu.SemaphoreType.DMA((1,)),
            pltpu.SemaphoreType.DMA((len(MIDS),)),
            pltpu.SemaphoreType.DMA((1,)),
            pltpu.SemaphoreType.DMA((1,)),
            pltpu.SemaphoreType.DMA((len(MIDS),)),
        ],
        compiler_params=pltpu.CompilerParams(collective_id=0),
    )(x, Wq, K_ext, V_ext, Wo)


# device time: 22193 ns/iter; 1.1236x vs baseline; 1.0285x over previous
import jax
import jax.numpy as jnp
from jax import lax
from jax.experimental import pallas as pl
from jax.experimental.pallas import tpu as pltpu

N_DEV = 8
B = 2
S_PER = 128
S_GLOB = N_DEV * S_PER
HQ = 4
DH = 64
G = HQ * B
S_NEAR = 3 * S_PER
LOCAL_WINDOW = 128
GLOBAL_PREFIX = 32
MIDS = (2, 3, 4, 5, 6)


def kernel(x, Wq, K_ext, V_ext, Wo):
    d_model = x.shape[-1]

    def body(x_ref, wq_ref, k_ref, v_ref, wo_ref, out_ref,
             nb_ref, q32_ref, k0_ref, part_ref, parts0_ref, aggin_ref,
             num32_ref, den32_ref,
             nbr_r_send, nbr_r_recv, nbr_l_send, nbr_l_recv,
             q32_send, q32_recv, k0_send, k0_recv,
             part_send, part_recv, agg_recv):
        my = lax.axis_index("i")
        left = lax.rem(my - 1 + N_DEV, N_DEV)
        right = lax.rem(my + 1, N_DEV)
        is_zero = my == 0
        is_mid = (my >= 2) & (my <= 6)
        is_leaf = (my == 2) | (my == 6)
        is_agg = (my == 3) | (my == 5)
        is_sender0 = (my >= 3) & (my <= 5)

        barrier_sem = pltpu.get_barrier_semaphore()
        for nbr in (left, right):
            pl.semaphore_signal(
                barrier_sem, inc=1,
                device_id=(nbr,), device_id_type=pl.DeviceIdType.MESH,
            )

        @pl.when(is_zero)
        def _():
            for d in MIDS:
                pl.semaphore_signal(
                    barrier_sem, inc=1,
                    device_id=(d,), device_id_type=pl.DeviceIdType.MESH,
                )

        @pl.when(is_mid)
        def _():
            pl.semaphore_signal(
                barrier_sem, inc=1,
                device_id=(0,), device_id_type=pl.DeviceIdType.MESH,
            )

        pl.semaphore_wait(barrier_sem, 2)

        @pl.when(is_zero)
        def _():
            pl.semaphore_wait(barrier_sem, len(MIDS))

        @pl.when(is_mid)
        def _():
            pl.semaphore_wait(barrier_sem, 1)

        wqb = wq_ref[:].astype(jnp.bfloat16)

        q32_rdmas = []
        for i, dd in enumerate(MIDS):
            q32_rdmas.append(pltpu.make_async_remote_copy(
                src_ref=q32_ref, dst_ref=q32_ref,
                send_sem=q32_send.at[i], recv_sem=q32_recv.at[0],
                device_id=(dd,), device_id_type=pl.DeviceIdType.MESH,
            ))

        @pl.when(is_zero)
        def _():
            q32 = lax.dot_general(
                x_ref[:, :GLOBAL_PREFIX, :].astype(jnp.bfloat16), wqb,
                (((2,), (0,)), ((), ())),
                preferred_element_type=jnp.float32,
            )
            q32_ref[:] = jnp.concatenate(
                [q32[:, :, h * DH:(h + 1) * DH] for h in range(HQ)], axis=0
            ).astype(jnp.bfloat16)
            for r in q32_rdmas:
                r.start()

        for j in range(HQ):
            nb_ref[pl.ds(j * B, B), pl.ds(S_PER, S_PER), :] = (
                k_ref[:, :, j, :].astype(jnp.bfloat16))
            nb_ref[pl.ds(G + j * B, B), pl.ds(S_PER, S_PER), :] = (
                v_ref[:, :, j, :].astype(jnp.bfloat16))

        @pl.when(jnp.logical_not(is_mid))
        def _():
            k0_ref[:] = jnp.zeros((2 * G, GLOBAL_PREFIX, DH), jnp.bfloat16)

        send_r = pltpu.make_async_remote_copy(
            src_ref=nb_ref.at[:, pl.ds(S_PER, S_PER), :],
            dst_ref=nb_ref.at[:, pl.ds(0, S_PER), :],
            send_sem=nbr_r_send.at[0], recv_sem=nbr_r_recv.at[0],
            device_id=(right,), device_id_type=pl.DeviceIdType.MESH,
        )
        send_l = pltpu.make_async_remote_copy(
            src_ref=nb_ref.at[:, pl.ds(S_PER, S_PER), :],
            dst_ref=nb_ref.at[:, pl.ds(2 * S_PER, S_PER), :],
            send_sem=nbr_l_send.at[0], recv_sem=nbr_l_recv.at[0],
            device_id=(left,), device_id_type=pl.DeviceIdType.MESH,
        )
        xb = x_ref[:].astype(jnp.bfloat16)

        k0_rdmas = []
        for i, dd in enumerate(MIDS):
            k0_rdmas.append(pltpu.make_async_remote_copy(
                src_ref=nb_ref.at[:, pl.ds(S_PER, GLOBAL_PREFIX), :],
                dst_ref=k0_ref,
                send_sem=k0_send.at[i], recv_sem=k0_recv.at[0],
                device_id=(dd,), device_id_type=pl.DeviceIdType.MESH,
            ))

        @pl.when(is_zero)
        def _():
            for r in k0_rdmas:
                r.start()

        send_r.start()
        send_l.start()

        q = lax.dot_general(
            xb, wqb, (((2,), (0,)), ((), ())),
            preferred_element_type=jnp.float32,
        )
        qall = jnp.concatenate(
            [q[:, :, h * DH:(h + 1) * DH] for h in range(HQ)], axis=0
        ).astype(jnp.bfloat16)

        leaf_target = jnp.where(my == 2, 3, 5)
        leaf_rdma = pltpu.make_async_remote_copy(
            src_ref=part_ref, dst_ref=aggin_ref,
            send_sem=part_send.at[0], recv_sem=agg_recv.at[0],
            device_id=(leaf_target,), device_id_type=pl.DeviceIdType.MESH,
        )
        slot0 = jnp.clip(my - 3, 0, 2)
        to0_rdma = pltpu.make_async_remote_copy(
            src_ref=part_ref, dst_ref=parts0_ref.at[slot0],
            send_sem=part_send.at[0], recv_sem=part_recv.at[slot0],
            device_id=(0,), device_id_type=pl.DeviceIdType.MESH,
        )

        @pl.when(is_mid)
        def _():
            q32_rdmas[0].wait_recv()
            kb_own = nb_ref[0:G, pl.ds(S_PER, S_PER), :]
            vb_own = nb_ref[G:2 * G, pl.ds(S_PER, S_PER), :]
            s32 = lax.dot_general(
                q32_ref[:], kb_own, (((2,), (2,)), ((0,), (0,))),
                preferred_element_type=jnp.float32,
            ) * 0.125
            e32 = jnp.exp(s32)
            p_num = lax.dot_general(
                e32.astype(jnp.bfloat16), vb_own,
                (((2,), (1,)), ((0,), (0,))),
                preferred_element_type=jnp.float32,
            )
            p_den = jnp.sum(e32, axis=-1, keepdims=True)
            part_ref[0] = p_num.astype(jnp.bfloat16)
            part_ref[1] = jnp.broadcast_to(
                p_den, (G, GLOBAL_PREFIX, DH)).astype(jnp.bfloat16)

        @pl.when(is_leaf)
        def _():
            leaf_rdma.start()

        @pl.when(is_agg)
        def _():
            leaf_rdma.wait_recv()
            part_ref[0] = (part_ref[0].astype(jnp.float32)
                           + aggin_ref[0].astype(jnp.float32)
                           ).astype(jnp.bfloat16)
            part_ref[1] = (part_ref[1].astype(jnp.float32)
                           + aggin_ref[1].astype(jnp.float32)
                           ).astype(jnp.bfloat16)

        @pl.when(is_sender0)
        def _():
            to0_rdma.start()

        @pl.when(is_mid)
        def _():
            k0_rdmas[0].wait_recv()

        sb = lax.dot_general(
            qall, k0_ref[0:G], (((2,), (2,)), ((0,), (0,))),
            preferred_element_type=jnp.float32,
        ) * 0.125
        eb = jnp.where(is_mid, jnp.exp(sb), 0.0)
        num = lax.dot_general(
            eb.astype(jnp.bfloat16), k0_ref[G:2 * G],
            (((2,), (1,)), ((0,), (0,))),
            preferred_element_type=jnp.float32,
        )
        den = jnp.sum(eb, axis=-1, keepdims=True)

        send_r.wait_recv()
        send_l.wait_recv()

        qi_glob = (lax.broadcasted_iota(jnp.int32, (S_PER, S_NEAR), 0)
                   + my * S_PER)
        fake_ki = (lax.broadcasted_iota(jnp.int32, (S_PER, S_NEAR), 1)
                   + (my - 1) * S_PER)
        real_ki = lax.rem(fake_ki + S_GLOB, S_GLOB)
        mask = ((jnp.abs(qi_glob - real_ki) <= LOCAL_WINDOW)
                | (real_ki < GLOBAL_PREFIX) | (qi_glob < GLOBAL_PREFIX))
        maskf = mask.astype(jnp.float32)[None, :, :]

        kb = nb_ref[0:G]
        vb = nb_ref[G:2 * G]
        s = lax.dot_general(
            qall, kb, (((2,), (2,)), ((0,), (0,))),
            preferred_element_type=jnp.float32,
        ) * 0.125
        e = jnp.exp(s) * maskf
        num = num + lax.dot_general(
            e.astype(jnp.bfloat16), vb, (((2,), (1,)), ((0,), (0,))),
            preferred_element_type=jnp.float32,
        )
        den = den + jnp.sum(e, axis=-1, keepdims=True)

        @pl.when(is_zero)
        def _():
            for i in range(3):
                part_recv_desc = pltpu.make_async_remote_copy(
                    src_ref=part_ref, dst_ref=parts0_ref.at[i],
                    send_sem=part_send.at[0], recv_sem=part_recv.at[i],
                    device_id=(0,), device_id_type=pl.DeviceIdType.MESH,
                )
                part_recv_desc.wait_recv()
            n32 = parts0_ref[0, 0].astype(jnp.float32)
            d32 = parts0_ref[0, 1, :, :, 0:1].astype(jnp.float32)
            for i in range(1, 3):
                n32 = n32 + parts0_ref[i, 0].astype(jnp.float32)
                d32 = d32 + parts0_ref[i, 1, :, :, 0:1].astype(jnp.float32)
            num32_ref[:] = n32
            den32_ref[:] = d32

        @pl.when(jnp.logical_not(is_zero))
        def _():
            num32_ref[:] = jnp.zeros((G, GLOBAL_PREFIX, DH), jnp.float32)
            den32_ref[:] = jnp.zeros((G, GLOBAL_PREFIX, 1), jnp.float32)

        n = jnp.concatenate(
            [num[:, :GLOBAL_PREFIX, :] + num32_ref[:],
             num[:, GLOBAL_PREFIX:, :]], axis=1)
        d = jnp.concatenate(
            [den[:, :GLOBAL_PREFIX, :] + den32_ref[:],
             den[:, GLOBAL_PREFIX:, :]], axis=1)

        ctx = (n / d).astype(jnp.bfloat16)
        acc = jnp.zeros((B, S_PER, d_model), dtype=jnp.float32)
        for h in range(HQ):
            woh = wo_ref[h * DH:(h + 1) * DH, :].astype(jnp.bfloat16)
            acc = acc + lax.dot_general(
                ctx[h * B:(h + 1) * B], woh, (((2,), (0,)), ((), ())),
                preferred_element_type=jnp.float32,
            )
        out_ref[:] = acc

        send_r.wait_send()
        send_l.wait_send()

        @pl.when(is_zero)
        def _():
            for r in q32_rdmas:
                r.wait_send()
            for r in k0_rdmas:
                r.wait_send()

        @pl.when(is_leaf)
        def _():
            leaf_rdma.wait_send()

        @pl.when(is_sender0)
        def _():
            to0_rdma.wait_send()

    out_shape = jax.ShapeDtypeStruct((B, S_PER, d_model), jnp.float32)
    return pl.pallas_call(
        body,
        out_shape=out_shape,
        in_specs=[pl.BlockSpec(memory_space=pltpu.VMEM)] * 5,
        out_specs=pl.BlockSpec(memory_space=pltpu.VMEM),
        scratch_shapes=[
            pltpu.VMEM((2 * G, S_NEAR, DH), jnp.bfloat16),
            pltpu.VMEM((G, GLOBAL_PREFIX, DH), jnp.bfloat16),
            pltpu.VMEM((2 * G, GLOBAL_PREFIX, DH), jnp.bfloat16),
            pltpu.VMEM((2, G, GLOBAL_PREFIX, DH), jnp.bfloat16),
            pltpu.VMEM((3, 2, G, GLOBAL_PREFIX, DH),
                       jnp.bfloat16),
            pltpu.VMEM((2, G, GLOBAL_PREFIX, DH), jnp.bfloat16),
            pltpu.VMEM((G, GLOBAL_PREFIX, DH), jnp.float32),
            pltpu.VMEM((G, GLOBAL_PREFIX, 1), jnp.float32),
            pltpu.SemaphoreType.DMA((1,)),
            pltpu.SemaphoreType.DMA((1,)),
            pltpu.SemaphoreType.DMA((1,)),
            pltpu.SemaphoreType.DMA((1,)),
            pltpu.SemaphoreType.DMA((len(MIDS),)),
            pltpu.SemaphoreType.DMA((1,)),
            pltpu.SemaphoreType.DMA((len(MIDS),)),
            pltpu.SemaphoreType.DMA((1,)),
            pltpu.SemaphoreType.DMA((1,)),
            pltpu.SemaphoreType.DMA((3,)),
            pltpu.SemaphoreType.DMA((1,)),
        ],
        compiler_params=pltpu.CompilerParams(collective_id=0),
    )(x, Wq, K_ext, V_ext, Wo)
